# baseline (speedup 1.0000x reference)
import jax
import jax.numpy as jnp
from jax.experimental import pallas as pl
from jax.experimental.pallas import tpu as pltpu

_BS = 2
_NQ = 1000
_NCTRL = 25
_T = 25
_VOC = 96
_NTGT = 20
_CLASS_W = 2.0
_COORD_W = 5.0
_TEXT_W = 2.0
_ALPHA = 0.25
_GAMMA = 2.0
_MATCH = 5
_NEG = -1e9
_U = _T


_UP = _U + 1
_R = _NTGT * _UP


_RE = _R + 8


def _ctc_kernel(x_ref, tgt_ref, out_ref, lse_s):
    def lse_body(t, _):
        x_t = x_ref[0, :, t, :]
        mm = jnp.max(x_t, axis=1, keepdims=True)
        lse = mm + jnp.log(jnp.sum(jnp.exp(x_t - mm), axis=1, keepdims=True))
        lse_s[pl.ds(t, 1), :] = jnp.transpose(lse)
        return 0

    jax.lax.fori_loop(0, _T, lse_body, 0)

    tgt = tgt_ref[0]
    iota_v = jax.lax.broadcasted_iota(jnp.int32, (_RE, _VOC + 1), 1)
    E = (tgt == iota_v).astype(jnp.float32)
    tgt_c = tgt[:_R]
    prev = jnp.concatenate([jnp.full((1, 1), -2, jnp.int32), tgt_c[:-1]], axis=0)
    allow = tgt_c != prev
    r_iota = jax.lax.broadcasted_iota(jnp.int32, (_R, 1), 0)
    is_u0 = (r_iota % _UP) == 0

    def lp(t):
        x_t = x_ref[0, :, t, :]
        raw = jax.lax.dot_general(
            E, x_t, (((1,), (1,)), ((), ())), preferred_element_type=jnp.float32
        )
        return raw - lse_s[pl.ds(t, 1), :]

    neg = jnp.float32(_NEG)
    neg_row = jnp.full((1, _NQ), _NEG, dtype=jnp.float32)

    lp0 = lp(0)
    alpha_c = jnp.where(is_u0, lp0[:_R], neg)
    alpha_b = jnp.where(is_u0, jnp.broadcast_to(lp0[_R : _R + 1], (_R, _NQ)), neg)

    def step(t, carry):
        ac, ab = carry
        lp_t = lp(t)
        lpc_t = lp_t[:_R]
        lpb_t = lp_t[_R : _R + 1]
        c_sh = jnp.concatenate([neg_row, ac[:-1]], axis=0)
        c_sh = jnp.where(is_u0, neg, c_sh)
        m_b = jnp.maximum(ab, c_sh)
        n_b = jnp.minimum(ab, c_sh)
        new_b = lpb_t + m_b + jnp.log1p(jnp.exp(n_b - m_b))
        a3 = jnp.where(allow, c_sh, neg)
        m_c = jnp.maximum(jnp.maximum(ac, ab), a3)
        s_c = jnp.exp(ac - m_c) + jnp.exp(ab - m_c) + jnp.exp(a3 - m_c)
        new_c = lpc_t + m_c + jnp.log(s_c)
        return (new_c, new_b)

    ac, ab = jax.lax.fori_loop(1, _T, step, (alpha_c, alpha_b))
    b_sh = jnp.concatenate([ab[1:], neg_row], axis=0)
    out_ref[0] = jnp.logaddexp(ac, b_sh) * jnp.float32(-1.0 / _U)


def _ctc_call(x_nat, tgt_stack):
    return pl.pallas_call(
        _ctc_kernel,
        grid=(_BS,),
        in_specs=[
            pl.BlockSpec((1, _NQ, _T, _VOC + 1), lambda b: (b, 0, 0, 0)),
            pl.BlockSpec((1, _RE, 1), lambda b: (b, 0, 0)),
        ],
        out_specs=pl.BlockSpec((1, _R, _NQ), lambda b: (b, 0, 0)),
        out_shape=jax.ShapeDtypeStruct((_BS, _R, _NQ), jnp.float32),
        scratch_shapes=[pltpu.VMEM((_T, _NQ), jnp.float32)],
        compiler_params=pltpu.CompilerParams(
            dimension_semantics=("parallel",),
        ),
    )(x_nat, tgt_stack)


_QB = 200


def _cc_kernel(lg_ref, pts_ref, tp_ref, out_ref):
    p = jax.nn.sigmoid(lg_ref[...])
    pos = _ALPHA * (1.0 - p) * (1.0 - p) * (-jnp.log(p + 1e-8))
    neg = (1.0 - _ALPHA) * p * p * (-jnp.log(1.0 - p + 1e-8))
    cc = jnp.mean(pos - neg, axis=1, keepdims=True)

    pts = pts_ref[...]
    col = jax.lax.broadcasted_iota(jnp.int32, (_QB, _BS * _NTGT), 1)
    acc = jnp.zeros((_QB, _BS * _NTGT), jnp.float32)
    for j in range(_BS * _NTGT):
        d = jnp.sum(jnp.abs(pts - tp_ref[j : j + 1, :]), axis=1, keepdims=True)
        acc = jnp.where(col == j, d, acc)
    out_ref[...] = _CLASS_W * cc + _COORD_W * acc


def _cc_call(lg2, pts2, tpts):
    nblk = (_BS * _NQ) // _QB
    return pl.pallas_call(
        _cc_kernel,
        grid=(nblk,),
        in_specs=[
            pl.BlockSpec((_QB, _NCTRL), lambda i: (i, 0)),
            pl.BlockSpec((_QB, 2 * _NCTRL), lambda i: (i, 0)),
            pl.BlockSpec((_BS * _NTGT, 2 * _NCTRL), lambda i: (0, 0)),
        ],
        out_specs=pl.BlockSpec((_QB, _BS * _NTGT), lambda i: (i, 0)),
        out_shape=jax.ShapeDtypeStruct((_BS * _NQ, _BS * _NTGT), jnp.float32),
    )(lg2, pts2, tpts)


def _match_kernel(ct_ref, tx_ref, cost_ref, idx_ref):
    c = ct_ref[0] + _TEXT_W * tx_ref[0]
    cost_ref[0] = c
    iq = jax.lax.broadcasted_iota(jnp.int32, (_NTGT, _NQ), 1)
    big = jnp.int32(1 << 30)
    for k in range(_MATCH):
        mn = jnp.min(c, axis=1, keepdims=True)
        idx = jnp.min(jnp.where(c == mn, iq, big), axis=1, keepdims=True)
        idx_ref[0, :, k : k + 1] = idx
        c = jnp.where(iq == idx, jnp.float32(3e38), c)


def _match_call(ctT, txT):
    return pl.pallas_call(
        _match_kernel,
        grid=(_BS,),
        in_specs=[
            pl.BlockSpec((1, _NTGT, _NQ), lambda b: (b, 0, 0)),
            pl.BlockSpec((1, _NTGT, _NQ), lambda b: (b, 0, 0)),
        ],
        out_specs=[
            pl.BlockSpec((1, _NTGT, _NQ), lambda b: (b, 0, 0)),
            pl.BlockSpec((1, _NTGT, _MATCH), lambda b: (b, 0, 0)),
        ],
        out_shape=[
            jax.ShapeDtypeStruct((_BS, _NTGT, _NQ), jnp.float32),
            jax.ShapeDtypeStruct((_BS, _NTGT, _MATCH), jnp.int32),
        ],
    )(ctT, txT)


def kernel(pred_logits, pred_ctrl_points, pred_text_logits, tgt_ctrl_points, tgt_texts):
    tgt_pad = jnp.pad(
        tgt_texts.astype(jnp.int32).reshape(_BS, _NTGT, _U),
        ((0, 0), (0, 0), (0, 1)),
        constant_values=-1,
    ).reshape(_BS, _R)
    extra = jnp.concatenate(
        [
            jnp.full((_BS, 1), _VOC, jnp.int32),
            jnp.full((_BS, _RE - _R - 1), -1, jnp.int32),
        ],
        axis=1,
    )
    tgt_stack = jnp.concatenate([tgt_pad, extra], axis=1).reshape(_BS, _RE, 1)
    lg2 = pred_logits.reshape(_BS * _NQ, _NCTRL)
    pts2 = pred_ctrl_points.reshape(_BS * _NQ, 2 * _NCTRL)
    tpts = tgt_ctrl_points.reshape(_BS * _NTGT, 2 * _NCTRL)

    text_full = _ctc_call(pred_text_logits, tgt_stack)
    text = text_full[:, _U - 1 :: _UP, :]
    C2 = _cc_call(lg2, pts2, tpts)

    C = C2.reshape(_BS, _NQ, _BS * _NTGT)
    ctT = jnp.stack(
        [C[b, :, b * _NTGT : (b + 1) * _NTGT].T for b in range(_BS)]
    )
    costT, idx = _match_call(ctT, text)
    cost = costT.transpose(0, 2, 1)
    src = idx.reshape(_BS, _NTGT * _MATCH)
    tgt_idx = jnp.broadcast_to(
        jnp.repeat(jnp.arange(_NTGT, dtype=jnp.int32), _MATCH), (_BS, _NTGT * _MATCH)
    )
    return (C, cost, src, tgt_idx)


# speedup vs baseline: 1.4394x; 1.4394x over previous
import jax
import jax.numpy as jnp
from jax.experimental import pallas as pl
from jax.experimental.pallas import tpu as pltpu

_BS = 2
_NQ = 1000
_NCTRL = 25
_T = 25
_VOC = 96
_NTGT = 20
_CLASS_W = 2.0
_COORD_W = 5.0
_TEXT_W = 2.0
_ALPHA = 0.25
_GAMMA = 2.0
_MATCH = 5
_NEG = -1e9
_U = _T


_UP = _U + 1
_R = _NTGT * _UP


_RE = _R + 8


def _ctc_kernel(x_ref, tgt_ref, out_ref, lse_s):
    def lse_body(t, _):
        x_t = x_ref[0, t]
        mm = jnp.max(x_t, axis=0, keepdims=True)
        lse_s[pl.ds(t, 1), :] = mm + jnp.log(
            jnp.sum(jnp.exp(x_t - mm), axis=0, keepdims=True)
        )
        return 0

    jax.lax.fori_loop(0, _T, lse_body, 0)

    tgt = tgt_ref[0]
    iota_v = jax.lax.broadcasted_iota(jnp.int32, (_RE, _VOC + 1), 1)
    E = (tgt == iota_v).astype(jnp.float32)
    tgt_c = tgt[:_R]
    prev = jnp.concatenate([jnp.full((1, 1), -2, jnp.int32), tgt_c[:-1]], axis=0)
    allow = tgt_c != prev
    r_iota = jax.lax.broadcasted_iota(jnp.int32, (_R, 1), 0)
    is_u0 = (r_iota % _UP) == 0

    def lp(t):
        x_t = x_ref[0, t]
        raw = jnp.dot(E, x_t, preferred_element_type=jnp.float32)
        return raw - lse_s[pl.ds(t, 1), :]

    neg = jnp.float32(_NEG)
    neg_row = jnp.full((1, _NQ), _NEG, dtype=jnp.float32)

    lp0 = lp(0)
    alpha_c = jnp.where(is_u0, lp0[:_R], neg)
    alpha_b = jnp.where(is_u0, jnp.broadcast_to(lp0[_R : _R + 1], (_R, _NQ)), neg)

    def step(t, carry):
        ac, ab = carry
        lp_t = lp(t)
        lpc_t = lp_t[:_R]
        lpb_t = lp_t[_R : _R + 1]
        c_sh = jnp.concatenate([neg_row, ac[:-1]], axis=0)
        c_sh = jnp.where(is_u0, neg, c_sh)
        m_b = jnp.maximum(ab, c_sh)
        n_b = jnp.minimum(ab, c_sh)
        new_b = lpb_t + m_b + jnp.log1p(jnp.exp(n_b - m_b))
        a3 = jnp.where(allow, c_sh, neg)
        m_c = jnp.maximum(jnp.maximum(ac, ab), a3)
        s_c = jnp.exp(ac - m_c) + jnp.exp(ab - m_c) + jnp.exp(a3 - m_c)
        new_c = lpc_t + m_c + jnp.log(s_c)
        return (new_c, new_b)

    ac, ab = jax.lax.fori_loop(1, _T, step, (alpha_c, alpha_b))
    b_sh = jnp.concatenate([ab[1:], neg_row], axis=0)
    out_ref[0] = jnp.logaddexp(ac, b_sh) * jnp.float32(-1.0 / _U)


def _ctc_call(x_nat, tgt_stack):
    return pl.pallas_call(
        _ctc_kernel,
        grid=(_BS,),
        in_specs=[
            pl.BlockSpec((1, _T, _VOC + 1, _NQ), lambda b: (b, 0, 0, 0)),
            pl.BlockSpec((1, _RE, 1), lambda b: (b, 0, 0)),
        ],
        out_specs=pl.BlockSpec((1, _R, _NQ), lambda b: (b, 0, 0)),
        out_shape=jax.ShapeDtypeStruct((_BS, _R, _NQ), jnp.float32),
        scratch_shapes=[pltpu.VMEM((_T, _NQ), jnp.float32)],
        compiler_params=pltpu.CompilerParams(
            dimension_semantics=("parallel",),
        ),
    )(x_nat, tgt_stack)


_QB = 200


def _cc_kernel(lg_ref, pts_ref, tp_ref, out_ref):
    p = jax.nn.sigmoid(lg_ref[...])
    pos = _ALPHA * (1.0 - p) * (1.0 - p) * (-jnp.log(p + 1e-8))
    neg = (1.0 - _ALPHA) * p * p * (-jnp.log(1.0 - p + 1e-8))
    cc = jnp.mean(pos - neg, axis=1, keepdims=True)

    pts = pts_ref[...]
    col = jax.lax.broadcasted_iota(jnp.int32, (_QB, _BS * _NTGT), 1)
    acc = jnp.zeros((_QB, _BS * _NTGT), jnp.float32)
    for j in range(_BS * _NTGT):
        d = jnp.sum(jnp.abs(pts - tp_ref[j : j + 1, :]), axis=1, keepdims=True)
        acc = jnp.where(col == j, d, acc)
    out_ref[...] = _CLASS_W * cc + _COORD_W * acc


def _cc_call(lg2, pts2, tpts):
    nblk = (_BS * _NQ) // _QB
    return pl.pallas_call(
        _cc_kernel,
        grid=(nblk,),
        in_specs=[
            pl.BlockSpec((_QB, _NCTRL), lambda i: (i, 0)),
            pl.BlockSpec((_QB, 2 * _NCTRL), lambda i: (i, 0)),
            pl.BlockSpec((_BS * _NTGT, 2 * _NCTRL), lambda i: (0, 0)),
        ],
        out_specs=pl.BlockSpec((_QB, _BS * _NTGT), lambda i: (i, 0)),
        out_shape=jax.ShapeDtypeStruct((_BS * _NQ, _BS * _NTGT), jnp.float32),
    )(lg2, pts2, tpts)


def _match_kernel(ct_ref, tx_ref, cost_ref, idx_ref):
    c = ct_ref[0] + _TEXT_W * tx_ref[0]
    cost_ref[0] = c
    iq = jax.lax.broadcasted_iota(jnp.int32, (_NTGT, _NQ), 1)
    big = jnp.int32(1 << 30)
    for k in range(_MATCH):
        mn = jnp.min(c, axis=1, keepdims=True)
        idx = jnp.min(jnp.where(c == mn, iq, big), axis=1, keepdims=True)
        idx_ref[0, :, k : k + 1] = idx
        c = jnp.where(iq == idx, jnp.float32(3e38), c)


def _match_call(ctT, txT):
    return pl.pallas_call(
        _match_kernel,
        grid=(_BS,),
        in_specs=[
            pl.BlockSpec((1, _NTGT, _NQ), lambda b: (b, 0, 0)),
            pl.BlockSpec((1, _NTGT, _NQ), lambda b: (b, 0, 0)),
        ],
        out_specs=[
            pl.BlockSpec((1, _NTGT, _NQ), lambda b: (b, 0, 0)),
            pl.BlockSpec((1, _NTGT, _MATCH), lambda b: (b, 0, 0)),
        ],
        out_shape=[
            jax.ShapeDtypeStruct((_BS, _NTGT, _NQ), jnp.float32),
            jax.ShapeDtypeStruct((_BS, _NTGT, _MATCH), jnp.int32),
        ],
    )(ctT, txT)


def kernel(pred_logits, pred_ctrl_points, pred_text_logits, tgt_ctrl_points, tgt_texts):
    tgt_pad = jnp.pad(
        tgt_texts.astype(jnp.int32).reshape(_BS, _NTGT, _U),
        ((0, 0), (0, 0), (0, 1)),
        constant_values=-1,
    ).reshape(_BS, _R)
    extra = jnp.concatenate(
        [
            jnp.full((_BS, 1), _VOC, jnp.int32),
            jnp.full((_BS, _RE - _R - 1), -1, jnp.int32),
        ],
        axis=1,
    )
    tgt_stack = jnp.concatenate([tgt_pad, extra], axis=1).reshape(_BS, _RE, 1)
    lg2 = pred_logits.reshape(_BS * _NQ, _NCTRL)
    pts2 = pred_ctrl_points.reshape(_BS * _NQ, 2 * _NCTRL)
    tpts = tgt_ctrl_points.reshape(_BS * _NTGT, 2 * _NCTRL)

    xT = pred_text_logits.transpose(0, 2, 3, 1)
    text_full = _ctc_call(xT, tgt_stack)
    text = text_full[:, _U - 1 :: _UP, :]
    C2 = _cc_call(lg2, pts2, tpts)

    C = C2.reshape(_BS, _NQ, _BS * _NTGT)
    ctT = jnp.stack(
        [C[b, :, b * _NTGT : (b + 1) * _NTGT].T for b in range(_BS)]
    )
    costT, idx = _match_call(ctT, text)
    cost = costT.transpose(0, 2, 1)
    src = idx.reshape(_BS, _NTGT * _MATCH)
    tgt_idx = jnp.broadcast_to(
        jnp.repeat(jnp.arange(_NTGT, dtype=jnp.int32), _MATCH), (_BS, _NTGT * _MATCH)
    )
    return (C, cost, src, tgt_idx)


# speedup vs baseline: 1.6192x; 1.1249x over previous
import jax
import jax.numpy as jnp
from jax.experimental import pallas as pl
from jax.experimental.pallas import tpu as pltpu

_BS = 2
_NQ = 1000
_NCTRL = 25
_T = 25
_VOC = 96
_NTGT = 20
_CLASS_W = 2.0
_COORD_W = 5.0
_TEXT_W = 2.0
_ALPHA = 0.25
_GAMMA = 2.0
_MATCH = 5
_NEG = -1e9
_U = _T


_UP = _U + 1
_R = _NTGT * _UP


_RE = _R + 8


def _ctc_kernel(x_ref, tgt_ref, out_ref, lse_s):
    def lse_body(t, _):
        x_t = x_ref[0, t]
        mm = jnp.max(x_t, axis=0, keepdims=True)
        lse_s[pl.ds(t, 1), :] = mm + jnp.log(
            jnp.sum(jnp.exp(x_t - mm), axis=0, keepdims=True)
        )
        return 0

    jax.lax.fori_loop(0, _T, lse_body, 0)

    tgt = tgt_ref[0]
    iota_v = jax.lax.broadcasted_iota(jnp.int32, (_RE, _VOC + 1), 1)
    E = (tgt == iota_v).astype(jnp.float32)
    tgt_c = tgt[:_R]
    prev = jnp.concatenate([jnp.full((1, 1), -2, jnp.int32), tgt_c[:-1]], axis=0)
    allow = tgt_c != prev
    r_iota = jax.lax.broadcasted_iota(jnp.int32, (_R, 1), 0)
    is_u0 = (r_iota % _UP) == 0

    def lp(t):
        x_t = x_ref[0, t]
        raw = jnp.dot(E, x_t, preferred_element_type=jnp.float32)
        return raw - lse_s[pl.ds(t, 1), :]

    neg = jnp.float32(_NEG)
    neg_row = jnp.full((1, _NQ), _NEG, dtype=jnp.float32)

    lp0 = lp(0)
    alpha_c = jnp.where(is_u0, lp0[:_R], neg)
    alpha_b = jnp.where(is_u0, jnp.broadcast_to(lp0[_R : _R + 1], (_R, _NQ)), neg)

    def step(t, carry):
        ac, ab = carry
        lp_t = lp(t)
        lpc_t = lp_t[:_R]
        lpb_t = lp_t[_R : _R + 1]
        c_sh = jnp.concatenate([neg_row, ac[:-1]], axis=0)
        c_sh = jnp.where(is_u0, neg, c_sh)
        m_b = jnp.maximum(ab, c_sh)
        n_b = jnp.minimum(ab, c_sh)
        new_b = lpb_t + m_b + jnp.log(jnp.exp(n_b - m_b) + 1.0)
        a3 = jnp.where(allow, c_sh, neg)
        m_c = jnp.maximum(jnp.maximum(ac, ab), a3)
        s_c = jnp.exp(ac - m_c) + jnp.exp(ab - m_c) + jnp.exp(a3 - m_c)
        new_c = lpc_t + m_c + jnp.log(s_c)
        return (new_c, new_b)

    ac, ab = jax.lax.fori_loop(1, _T, step, (alpha_c, alpha_b))
    b_sh = jnp.concatenate([ab[1:], neg_row], axis=0)
    out_ref[0] = jnp.logaddexp(ac, b_sh) * jnp.float32(-1.0 / _U)


def _ctc_call(x_nat, tgt_stack):
    return pl.pallas_call(
        _ctc_kernel,
        grid=(_BS,),
        in_specs=[
            pl.BlockSpec((1, _T, _VOC + 1, _NQ), lambda b: (b, 0, 0, 0)),
            pl.BlockSpec((1, _RE, 1), lambda b: (b, 0, 0)),
        ],
        out_specs=pl.BlockSpec((1, _R, _NQ), lambda b: (b, 0, 0)),
        out_shape=jax.ShapeDtypeStruct((_BS, _R, _NQ), jnp.float32),
        scratch_shapes=[pltpu.VMEM((_T, _NQ), jnp.float32)],
        compiler_params=pltpu.CompilerParams(
            dimension_semantics=("parallel",),
        ),
    )(x_nat, tgt_stack)


_QB = 200


def _cc_kernel(lg_ref, pts_ref, tp_ref, out_ref):
    p = jax.nn.sigmoid(lg_ref[...])
    pos = _ALPHA * (1.0 - p) * (1.0 - p) * (-jnp.log(p + 1e-8))
    neg = (1.0 - _ALPHA) * p * p * (-jnp.log(1.0 - p + 1e-8))
    cc = jnp.mean(pos - neg, axis=1, keepdims=True)

    pts = pts_ref[...]
    col = jax.lax.broadcasted_iota(jnp.int32, (_QB, _BS * _NTGT), 1)
    acc = jnp.zeros((_QB, _BS * _NTGT), jnp.float32)
    for j in range(_BS * _NTGT):
        d = jnp.sum(jnp.abs(pts - tp_ref[j : j + 1, :]), axis=1, keepdims=True)
        acc = jnp.where(col == j, d, acc)
    out_ref[...] = _CLASS_W * cc + _COORD_W * acc


def _cc_call(lg2, pts2, tpts):
    nblk = (_BS * _NQ) // _QB
    return pl.pallas_call(
        _cc_kernel,
        grid=(nblk,),
        in_specs=[
            pl.BlockSpec((_QB, _NCTRL), lambda i: (i, 0)),
            pl.BlockSpec((_QB, 2 * _NCTRL), lambda i: (i, 0)),
            pl.BlockSpec((_BS * _NTGT, 2 * _NCTRL), lambda i: (0, 0)),
        ],
        out_specs=pl.BlockSpec((_QB, _BS * _NTGT), lambda i: (i, 0)),
        out_shape=jax.ShapeDtypeStruct((_BS * _NQ, _BS * _NTGT), jnp.float32),
    )(lg2, pts2, tpts)


def _match_kernel(ct_ref, tx_ref, cost_ref, idx_ref):
    c = ct_ref[0] + _TEXT_W * tx_ref[0]
    cost_ref[0] = c
    iq = jax.lax.broadcasted_iota(jnp.int32, (_NTGT, _NQ), 1)
    big = jnp.int32(1 << 30)
    for k in range(_MATCH):
        mn = jnp.min(c, axis=1, keepdims=True)
        idx = jnp.min(jnp.where(c == mn, iq, big), axis=1, keepdims=True)
        idx_ref[0, :, k : k + 1] = idx
        c = jnp.where(iq == idx, jnp.float32(3e38), c)


def _match_call(ctT, txT):
    return pl.pallas_call(
        _match_kernel,
        grid=(_BS,),
        in_specs=[
            pl.BlockSpec((1, _NTGT, _NQ), lambda b: (b, 0, 0)),
            pl.BlockSpec((1, _NTGT, _NQ), lambda b: (b, 0, 0)),
        ],
        out_specs=[
            pl.BlockSpec((1, _NTGT, _NQ), lambda b: (b, 0, 0)),
            pl.BlockSpec((1, _NTGT, _MATCH), lambda b: (b, 0, 0)),
        ],
        out_shape=[
            jax.ShapeDtypeStruct((_BS, _NTGT, _NQ), jnp.float32),
            jax.ShapeDtypeStruct((_BS, _NTGT, _MATCH), jnp.int32),
        ],
    )(ctT, txT)


def kernel(pred_logits, pred_ctrl_points, pred_text_logits, tgt_ctrl_points, tgt_texts):
    tgt_pad = jnp.pad(
        tgt_texts.astype(jnp.int32).reshape(_BS, _NTGT, _U),
        ((0, 0), (0, 0), (0, 1)),
        constant_values=-1,
    ).reshape(_BS, _R)
    extra = jnp.concatenate(
        [
            jnp.full((_BS, 1), _VOC, jnp.int32),
            jnp.full((_BS, _RE - _R - 1), -1, jnp.int32),
        ],
        axis=1,
    )
    tgt_stack = jnp.concatenate([tgt_pad, extra], axis=1).reshape(_BS, _RE, 1)
    lg2 = pred_logits.reshape(_BS * _NQ, _NCTRL)
    pts2 = pred_ctrl_points.reshape(_BS * _NQ, 2 * _NCTRL)
    tpts = tgt_ctrl_points.reshape(_BS * _NTGT, 2 * _NCTRL)

    xT = pred_text_logits.transpose(0, 2, 3, 1)
    text_full = _ctc_call(xT, tgt_stack)
    text = text_full[:, _U - 1 :: _UP, :]
    C2 = _cc_call(lg2, pts2, tpts)

    C = C2.reshape(_BS, _NQ, _BS * _NTGT)
    ctT = jnp.stack(
        [C[b, :, b * _NTGT : (b + 1) * _NTGT].T for b in range(_BS)]
    )
    costT, idx = _match_call(ctT, text)
    cost = costT.transpose(0, 2, 1)
    src = idx.reshape(_BS, _NTGT * _MATCH)
    tgt_idx = jnp.broadcast_to(
        jnp.repeat(jnp.arange(_NTGT, dtype=jnp.int32), _MATCH), (_BS, _NTGT * _MATCH)
    )
    return (C, cost, src, tgt_idx)


# speedup vs baseline: 1.6297x; 1.0065x over previous
import jax
import jax.numpy as jnp
from jax.experimental import pallas as pl
from jax.experimental.pallas import tpu as pltpu

_BS = 2
_NQ = 1000
_NCTRL = 25
_T = 25
_VOC = 96
_NTGT = 20
_CLASS_W = 2.0
_COORD_W = 5.0
_TEXT_W = 2.0
_ALPHA = 0.25
_GAMMA = 2.0
_MATCH = 5
_NEG = -1e9
_U = _T


_UP = _U + 1
_R = _NTGT * _UP


_RE = _R + 8


def _ctc_kernel(x_ref, tgt_ref, out_ref, lse_s):
    def lse_body(t, _):
        x_t = x_ref[0, t]
        mm = jnp.max(x_t, axis=0, keepdims=True)
        lse_s[pl.ds(t, 1), :] = mm + jnp.log(
            jnp.sum(jnp.exp(x_t - mm), axis=0, keepdims=True)
        )
        return 0

    jax.lax.fori_loop(0, _T, lse_body, 0)

    tgt = tgt_ref[0]
    iota_v = jax.lax.broadcasted_iota(jnp.int32, (_RE, _VOC + 1), 1)
    E = (tgt == iota_v).astype(jnp.float32)
    tgt_c = tgt[:_R]
    prev = jnp.concatenate([jnp.full((1, 1), -2, jnp.int32), tgt_c[:-1]], axis=0)
    allow = tgt_c != prev
    r_iota = jax.lax.broadcasted_iota(jnp.int32, (_R, 1), 0)
    is_u0 = (r_iota % _UP) == 0

    def lp(t):
        x_t = x_ref[0, t]
        raw = jnp.dot(E, x_t, preferred_element_type=jnp.float32)
        return raw - lse_s[pl.ds(t, 1), :]

    neg = jnp.float32(_NEG)
    neg_row = jnp.full((1, _NQ), _NEG, dtype=jnp.float32)

    lp0 = lp(0)
    alpha_c = jnp.where(is_u0, lp0[:_R], neg)
    alpha_b = jnp.where(is_u0, jnp.broadcast_to(lp0[_R : _R + 1], (_R, _NQ)), neg)

    def step(t, carry):
        ac, ab = carry
        lp_t = lp(t)
        lpc_t = lp_t[:_R]
        lpb_t = lp_t[_R : _R + 1]
        c_sh = jnp.concatenate([neg_row, ac[:-1]], axis=0)
        c_sh = jnp.where(is_u0, neg, c_sh)
        m_b = jnp.maximum(ab, c_sh)
        n_b = jnp.minimum(ab, c_sh)
        new_b = lpb_t + m_b + jnp.log(jnp.exp(n_b - m_b) + 1.0)
        a3 = jnp.where(allow, c_sh, neg)
        m_c = jnp.maximum(jnp.maximum(ac, ab), a3)
        s_c = jnp.exp(ac - m_c) + jnp.exp(ab - m_c) + jnp.exp(a3 - m_c)
        new_c = lpc_t + m_c + jnp.log(s_c)
        return (new_c, new_b)

    ac, ab = jax.lax.fori_loop(1, _T, step, (alpha_c, alpha_b))
    b_sh = jnp.concatenate([ab[1:], neg_row], axis=0)
    X = jnp.logaddexp(ac, b_sh) * jnp.float32(-1.0 / _U)
    row_n = jax.lax.broadcasted_iota(jnp.int32, (_NTGT, _R), 0)
    col_r = jax.lax.broadcasted_iota(jnp.int32, (_NTGT, _R), 1)
    sel = (col_r == _UP * row_n + (_U - 1)).astype(jnp.float32)
    out_ref[0] = jnp.dot(sel, X, preferred_element_type=jnp.float32)


def _ctc_call(x_nat, tgt_stack):
    return pl.pallas_call(
        _ctc_kernel,
        grid=(_BS,),
        in_specs=[
            pl.BlockSpec((1, _T, _VOC + 1, _NQ), lambda b: (b, 0, 0, 0)),
            pl.BlockSpec((1, _RE, 1), lambda b: (b, 0, 0)),
        ],
        out_specs=pl.BlockSpec((1, _NTGT, _NQ), lambda b: (b, 0, 0)),
        out_shape=jax.ShapeDtypeStruct((_BS, _NTGT, _NQ), jnp.float32),
        scratch_shapes=[pltpu.VMEM((_T, _NQ), jnp.float32)],
        compiler_params=pltpu.CompilerParams(
            dimension_semantics=("arbitrary",),
        ),
    )(x_nat, tgt_stack)


_QB = 200


def _cc_kernel(lg_ref, pts_ref, tp_ref, out_ref):
    p = jax.nn.sigmoid(lg_ref[...])
    pos = _ALPHA * (1.0 - p) * (1.0 - p) * (-jnp.log(p + 1e-8))
    neg = (1.0 - _ALPHA) * p * p * (-jnp.log(1.0 - p + 1e-8))
    cc = jnp.mean(pos - neg, axis=1, keepdims=True)

    pts = pts_ref[...]
    col = jax.lax.broadcasted_iota(jnp.int32, (_QB, _BS * _NTGT), 1)
    acc = jnp.zeros((_QB, _BS * _NTGT), jnp.float32)
    for j in range(_BS * _NTGT):
        d = jnp.sum(jnp.abs(pts - tp_ref[j : j + 1, :]), axis=1, keepdims=True)
        acc = jnp.where(col == j, d, acc)
    out_ref[...] = _CLASS_W * cc + _COORD_W * acc


def _cc_call(lg2, pts2, tpts):
    nblk = (_BS * _NQ) // _QB
    return pl.pallas_call(
        _cc_kernel,
        grid=(nblk,),
        in_specs=[
            pl.BlockSpec((_QB, _NCTRL), lambda i: (i, 0)),
            pl.BlockSpec((_QB, 2 * _NCTRL), lambda i: (i, 0)),
            pl.BlockSpec((_BS * _NTGT, 2 * _NCTRL), lambda i: (0, 0)),
        ],
        out_specs=pl.BlockSpec((_QB, _BS * _NTGT), lambda i: (i, 0)),
        out_shape=jax.ShapeDtypeStruct((_BS * _NQ, _BS * _NTGT), jnp.float32),
    )(lg2, pts2, tpts)


def _match_kernel(csl_ref, tx_ref, cost_ref, idx_ref):
    ct = jnp.transpose(csl_ref[0])
    c = ct + _TEXT_W * tx_ref[0]
    cost_ref[0] = jnp.transpose(c)
    iq = jax.lax.broadcasted_iota(jnp.int32, (_NTGT, _NQ), 1)
    big = jnp.int32(1 << 30)
    for k in range(_MATCH):
        mn = jnp.min(c, axis=1, keepdims=True)
        idx = jnp.min(jnp.where(c == mn, iq, big), axis=1, keepdims=True)
        idx_ref[0, :, k : k + 1] = idx
        c = jnp.where(iq == idx, jnp.float32(3e38), c)


def _match_call(csl, txT):
    return pl.pallas_call(
        _match_kernel,
        grid=(_BS,),
        in_specs=[
            pl.BlockSpec((1, _NQ, _NTGT), lambda b: (b, 0, 0)),
            pl.BlockSpec((1, _NTGT, _NQ), lambda b: (b, 0, 0)),
        ],
        out_specs=[
            pl.BlockSpec((1, _NQ, _NTGT), lambda b: (b, 0, 0)),
            pl.BlockSpec((1, _NTGT, _MATCH), lambda b: (b, 0, 0)),
        ],
        out_shape=[
            jax.ShapeDtypeStruct((_BS, _NQ, _NTGT), jnp.float32),
            jax.ShapeDtypeStruct((_BS, _NTGT, _MATCH), jnp.int32),
        ],
    )(csl, txT)


def kernel(pred_logits, pred_ctrl_points, pred_text_logits, tgt_ctrl_points, tgt_texts):
    tgt_pad = jnp.pad(
        tgt_texts.astype(jnp.int32).reshape(_BS, _NTGT, _U),
        ((0, 0), (0, 0), (0, 1)),
        constant_values=-1,
    ).reshape(_BS, _R)
    extra = jnp.concatenate(
        [
            jnp.full((_BS, 1), _VOC, jnp.int32),
            jnp.full((_BS, _RE - _R - 1), -1, jnp.int32),
        ],
        axis=1,
    )
    tgt_stack = jnp.concatenate([tgt_pad, extra], axis=1).reshape(_BS, _RE, 1)
    lg2 = pred_logits.reshape(_BS * _NQ, _NCTRL)
    pts2 = pred_ctrl_points.reshape(_BS * _NQ, 2 * _NCTRL)
    tpts = tgt_ctrl_points.reshape(_BS * _NTGT, 2 * _NCTRL)

    xT = pred_text_logits.transpose(0, 2, 3, 1)
    text = _ctc_call(xT, tgt_stack)
    C2 = _cc_call(lg2, pts2, tpts)

    C = C2.reshape(_BS, _NQ, _BS * _NTGT)
    csl = jnp.stack([C[b, :, b * _NTGT : (b + 1) * _NTGT] for b in range(_BS)])
    cost, idx = _match_call(csl, text)
    src = idx.reshape(_BS, _NTGT * _MATCH)
    tgt_idx = jnp.broadcast_to(
        jnp.repeat(jnp.arange(_NTGT, dtype=jnp.int32), _MATCH), (_BS, _NTGT * _MATCH)
    )
    return (C, cost, src, tgt_idx)
